# baseline (speedup 1.0000x reference)
import functools

import jax
import jax.numpy as jnp
import numpy as np
from jax import lax
from jax.experimental import pallas as pl
from jax.experimental.pallas import tpu as pltpu
from jax.experimental.pallas import tpu_sc as plsc

N = 10000
G = 64
DIN = 128
H = 64
H2 = 32
OUT = 2

NPAD = 10112
ZR = NPAD // 16
C = 128
NC, NS = 2, 16
EPS = 1e-5


def _mesh():
    return plsc.VectorSubcoreMesh(core_axis_name="c", subcore_axis_name="s")


def _make_deg_kernel(KA, KB, SLOW):
    DW = 16
    K = KA + KB

    @functools.partial(
        pl.kernel,
        out_type=jax.ShapeDtypeStruct((NC, NPAD, DW), jnp.float32),
        mesh=_mesh(),
        compiler_params=pltpu.CompilerParams(use_tc_tiling_on_sc=False),
        scratch_types=[
            pltpu.VMEM((K, C), jnp.int32),
            pltpu.VMEM((C, DW), jnp.float32),
            pltpu.VMEM_SHARED((NPAD, DW), jnp.float32),
        ],
    )
    def deg_kernel(ea_hbm, eb_hbm, ones_hbm, zeros_hbm, out_hbm,
                   dst_v, ones_v, acc_sh):
        c = lax.axis_index("c")
        s = lax.axis_index("s")
        kmax = jnp.where(c == SLOW, KA, K)
        pltpu.sync_copy(zeros_hbm.at[pl.ds(s * ZR, ZR)],
                        acc_sh.at[pl.ds(s * ZR, ZR)])
        pltpu.sync_copy(ea_hbm.at[1, :, c, s], dst_v.at[pl.ds(0, KA)])
        pltpu.sync_copy(eb_hbm.at[1, :, c, s], dst_v.at[pl.ds(KA, KB)])
        pltpu.sync_copy(ones_hbm, ones_v)
        plsc.subcore_barrier()

        def body(j, carry):
            @pl.when(j < kmax)
            def _():
                pltpu.sync_copy(ones_v, acc_sh.at[dst_v.at[j]], add=True)
            return carry

        lax.fori_loop(0, K, body, 0)
        plsc.subcore_barrier()
        pltpu.sync_copy(acc_sh.at[pl.ds(s * ZR, ZR)],
                        out_hbm.at[c].at[pl.ds(s * ZR, ZR)])

    return deg_kernel


def _make_prop_kernel(KA, KB, SLOW, W):
    K = KA + KB

    @functools.partial(
        pl.kernel,
        out_type=jax.ShapeDtypeStruct((NC, NPAD, W), jnp.float32),
        mesh=_mesh(),
        compiler_params=pltpu.CompilerParams(use_tc_tiling_on_sc=False),
        scratch_types=[
            pltpu.VMEM((K, C), jnp.int32),
            pltpu.VMEM((K, C), jnp.int32),
            pltpu.VMEM((C, W), jnp.float32),
            pltpu.VMEM((C, W), jnp.float32),
            pltpu.VMEM_SHARED((NPAD, W), jnp.float32),
            pltpu.SemaphoreType.DMA,
            pltpu.SemaphoreType.DMA,
        ],
    )
    def prop_kernel(hs_hbm, ea_hbm, eb_hbm, zeros_hbm, out_hbm,
                    src_v, dst_v, rows_a, rows_b, acc_sh, sem_a, sem_b):
        c = lax.axis_index("c")
        s = lax.axis_index("s")
        kmax = jnp.where(c == SLOW, KA, K)
        pltpu.sync_copy(zeros_hbm.at[pl.ds(s * ZR, ZR)],
                        acc_sh.at[pl.ds(s * ZR, ZR)])
        pltpu.sync_copy(ea_hbm.at[0, :, c, s], src_v.at[pl.ds(0, KA)])
        pltpu.sync_copy(eb_hbm.at[0, :, c, s], src_v.at[pl.ds(KA, KB)])
        pltpu.sync_copy(ea_hbm.at[1, :, c, s], dst_v.at[pl.ds(0, KA)])
        pltpu.sync_copy(eb_hbm.at[1, :, c, s], dst_v.at[pl.ds(KA, KB)])
        plsc.subcore_barrier()

        pltpu.async_copy(hs_hbm.at[src_v.at[0]], rows_a, sem_a)

        def body2(i, carry):
            j = 2 * i

            @pl.when(j + 1 < kmax)
            def _():
                pltpu.async_copy(hs_hbm.at[src_v.at[j + 1]], rows_b, sem_b)

            @pl.when(j < kmax)
            def _():
                pltpu.make_async_copy(
                    hs_hbm.at[src_v.at[j]], rows_a, sem_a).wait()
                pltpu.sync_copy(rows_a, acc_sh.at[dst_v.at[j]], add=True)

            @pl.when(j + 2 < kmax)
            def _():
                pltpu.async_copy(hs_hbm.at[src_v.at[j + 2]], rows_a, sem_a)

            @pl.when(j + 1 < kmax)
            def _():
                pltpu.make_async_copy(
                    hs_hbm.at[src_v.at[j + 1]], rows_b, sem_b).wait()
                pltpu.sync_copy(rows_b, acc_sh.at[dst_v.at[j + 1]], add=True)

            return carry

        lax.fori_loop(0, (K + 1) // 2, body2, 0)
        plsc.subcore_barrier()
        pltpu.sync_copy(acc_sh.at[pl.ds(s * ZR, ZR)],
                        out_hbm.at[c].at[pl.ds(s * ZR, ZR)])

    return prop_kernel


def _row_mask():
    rows = lax.broadcasted_iota(jnp.int32, (NPAD, 1), 0)
    return rows < N


def _stage_a0(x_ref, w_ref, u_ref):
    u_ref[...] = jnp.dot(x_ref[...], w_ref[...],
                         preferred_element_type=jnp.float32)


def _stage_a1(u_ref, degp_ref, hs_ref, dis_ref):
    deg = degp_ref[0][:, 0:1] + degp_ref[1][:, 0:1]
    valid = _row_mask() & (deg > 0.0)
    dis = jnp.where(valid, lax.rsqrt(jnp.maximum(deg, 1e-20)), 0.0)
    dis_ref[...] = dis
    hs_ref[...] = u_ref[...] * dis


def _stage_b(narrow_out):
    def body(accq_ref, dis2_ref, b_ref, g_ref, be_ref, wblk_ref, hs_ref):
        dis2 = dis2_ref[...]
        mf = (dis2 > 0.0).astype(jnp.float32)
        h = ((accq_ref[0] + accq_ref[1]) * dis2 + b_ref[...]) * mf
        s = jnp.sum(h, axis=0, keepdims=True)
        mu = jnp.concatenate([(s[:, 0:H] + s[:, H:2 * H]) * (1.0 / N)] * 2,
                             axis=1)
        d = (h - mu) * mf
        v = jnp.sum(d * d, axis=0, keepdims=True)
        var = jnp.concatenate([(v[:, 0:H] + v[:, H:2 * H]) * (1.0 / N)] * 2,
                              axis=1)
        y = (h - mu) * lax.rsqrt(var + EPS) * g_ref[...] + be_ref[...]
        y = jnp.maximum(y, 0.0) * mf
        out = jnp.dot(y, wblk_ref[...], preferred_element_type=jnp.float32)
        if narrow_out:
            disn = jnp.concatenate([dis2[:, 0:H2], dis2[:, H:H + H2]], axis=1)
            hs_ref[...] = out * disn
        else:
            hs_ref[...] = out * dis2

    return body


def _c4(v, op):
    return op(op(v[:, 0:H2], v[:, H2:2 * H2]),
              op(v[:, 2 * H2:3 * H2], v[:, 3 * H2:4 * H2]))


def _stage_c(accq_ref, disq_ref, b4_ref, g4_ref, be4_ref, bid128_ref,
             bid_ref, wc1a_ref, wc1b_ref, bc1_ref, wc2_ref, bc2_ref,
             out_ref, sums_ref, mx_ref):
    b128 = bid128_ref[...]
    mf = (b128 < G).astype(jnp.float32)
    h = ((accq_ref[0] + accq_ref[1]) * disq_ref[...] + b4_ref[...]) * mf
    s = jnp.sum(h, axis=0, keepdims=True)
    mu = jnp.concatenate([_c4(s, jnp.add) * (1.0 / N)] * 4, axis=1)
    d = (h - mu) * mf
    v = jnp.sum(d * d, axis=0, keepdims=True)
    var = jnp.concatenate([_c4(v, jnp.add) * (1.0 / N)] * 4, axis=1)
    y = (h - mu) * lax.rsqrt(var + EPS) * g4_ref[...] + be4_ref[...]
    y = jnp.maximum(y, 0.0) * mf

    bid = bid_ref[...]
    oh = (bid == lax.broadcasted_iota(jnp.int32, (1, G), 1)).astype(jnp.float32)
    mcol = (bid < G).astype(jnp.float32)
    counts = lax.dot_general(oh, mcol, (((0,), (0,)), ((), ())),
                             preferred_element_type=jnp.float32)

    def gbody(g, carry):
        ym = jnp.where(b128 == g, y, 0.0)
        sg = jnp.sum(ym, axis=0, keepdims=True)
        mg = jnp.max(ym, axis=0, keepdims=True)
        sums_ref[pl.ds(g, 1), :] = _c4(sg, jnp.add)
        mx_ref[pl.ds(g, 1), :] = _c4(mg, jnp.maximum)
        return carry

    lax.fori_loop(0, G, gbody, 0)
    mean = sums_ref[...] / jnp.maximum(counts, 1.0)
    mx = mx_ref[...]

    zz = (lax.dot_general(mean, wc1a_ref[...], (((1,), (0,)), ((), ())),
                          preferred_element_type=jnp.float32)
          + lax.dot_general(mx, wc1b_ref[...], (((1,), (0,)), ((), ())),
                            preferred_element_type=jnp.float32)
          + bc1_ref[...])
    zz = jnp.maximum(zz, 0.0)
    out_ref[...] = lax.dot_general(zz, wc2_ref[...], (((1,), (0,)), ((), ())),
                                   preferred_element_type=jnp.float32) + bc2_ref[...]


def kernel(x, edge_index, batch, W1, b1, g1, be1, W2, b2, g2, be2,
           W3, b3, g3, be3, Wc1, bc1, Wc2, bc2):
    E = edge_index.shape[1]
    ET = E + N

    KA, KB, SLOW = 66, 30, 1
    NEA = NC * NS * KA * C
    NEB = NS * KB * C
    PADLEN = NEA + NEB - ET

    ei = edge_index.astype(jnp.int32)
    sl = jnp.arange(N, dtype=jnp.int32)
    loops = jnp.stack([sl, sl])
    padblk = jnp.stack([jnp.zeros((PADLEN,), jnp.int32),
                        N + (jnp.arange(PADLEN, dtype=jnp.int32) %
                             (NPAD - N))])
    alle = jnp.concatenate([ei, loops, padblk], axis=1)
    eA = alle[:, :NEA].reshape(2, KA, NC, NS, C)
    eBf = alle[:, NEA:].reshape(2, KB, NS, C)
    eB = jnp.stack([jnp.zeros_like(eBf), eBf], axis=2) if SLOW == 0 \
        else jnp.stack([eBf, jnp.zeros_like(eBf)], axis=2)

    xp = jnp.pad(x, ((0, NPAD - N), (0, 0)))
    bidp = jnp.pad(batch.astype(jnp.int32), (0, NPAD - N),
                   constant_values=G).reshape(NPAD, 1)

    ones16 = jnp.ones((C, 16), jnp.float32)
    zeros16 = jnp.zeros((NPAD, 16), jnp.float32)
    zeros64 = jnp.zeros((NPAD, H), jnp.float32)
    zeros32 = jnp.zeros((NPAD, H2), jnp.float32)

    degp = _make_deg_kernel(KA, KB, SLOW)(eA, eB, ones16, zeros16)
    u1 = pl.pallas_call(
        _stage_a0,
        out_shape=jax.ShapeDtypeStruct((NPAD, H), jnp.float32),
    )(xp, W1)

    hs1, dis = pl.pallas_call(
        _stage_a1,
        out_shape=(jax.ShapeDtypeStruct((NPAD, H), jnp.float32),
                   jax.ShapeDtypeStruct((NPAD, 1), jnp.float32)),
    )(u1, degp)

    prop64 = _make_prop_kernel(KA, KB, SLOW, H)
    prop32 = _make_prop_kernel(KA, KB, SLOW, H2)

    NP2 = NPAD // 2
    dis2 = jnp.broadcast_to(dis, (NPAD, H)).reshape(NP2, 2 * H)
    zb = jnp.zeros((H, H), jnp.float32)
    w2blk = jnp.concatenate(
        [jnp.concatenate([W2, zb], 1), jnp.concatenate([zb, W2], 1)], 0)
    zb2 = jnp.zeros((H, H2), jnp.float32)
    w3blk = jnp.concatenate(
        [jnp.concatenate([W3, zb2], 1), jnp.concatenate([zb2, W3], 1)], 0)

    def b_call(accp, b, g, be, wblk, wout, narrow):
        return pl.pallas_call(
            _stage_b(narrow),
            out_shape=jax.ShapeDtypeStruct((NP2, wout), jnp.float32),
        )(accp.reshape(NC, NP2, 2 * H), dis2,
          jnp.tile(b.reshape(1, H), (1, 2)), jnp.tile(g.reshape(1, H), (1, 2)),
          jnp.tile(be.reshape(1, H), (1, 2)), wblk)

    accp1 = prop64(hs1, eA, eB, zeros64)
    hs2 = b_call(accp1, b1, g1, be1, w2blk, 2 * H, False).reshape(NPAD, H)

    accp2 = prop64(hs2, eA, eB, zeros64)
    hs3 = b_call(accp2, b2, g2, be2, w3blk, 2 * H2, True).reshape(NPAD, H2)

    accp3 = prop32(hs3, eA, eB, zeros32)

    NQ = NPAD // 4
    accq3 = accp3.reshape(NC, NQ, 4 * H2)
    disq = jnp.broadcast_to(dis, (NPAD, H2)).reshape(NQ, 4 * H2)
    bid128 = jnp.broadcast_to(bidp, (NPAD, H2)).reshape(NQ, 4 * H2)
    b4 = jnp.tile(b3.reshape(1, H2), (1, 4))
    g4 = jnp.tile(g3.reshape(1, H2), (1, 4))
    be4 = jnp.tile(be3.reshape(1, H2), (1, 4))

    out = pl.pallas_call(
        _stage_c,
        out_shape=jax.ShapeDtypeStruct((G, OUT), jnp.float32),
        scratch_shapes=[pltpu.VMEM((G, H2), jnp.float32),
                        pltpu.VMEM((G, H2), jnp.float32)],
    )(accq3, disq, b4, g4, be4, bid128, bidp,
      Wc1[:H2], Wc1[H2:], bc1.reshape(1, H2), Wc2, bc2.reshape(1, OUT))

    return out


# speedup vs baseline: 1.0597x; 1.0597x over previous
import functools

import jax
import jax.numpy as jnp
import numpy as np
from jax import lax
from jax.experimental import pallas as pl
from jax.experimental.pallas import tpu as pltpu
from jax.experimental.pallas import tpu_sc as plsc

N = 10000
G = 64
DIN = 128
H = 64
H2 = 32
OUT = 2

NPAD = 10112
ZR = NPAD // 16
C = 128
NC, NS = 2, 16
EPS = 1e-5


def _mesh():
    return plsc.VectorSubcoreMesh(core_axis_name="c", subcore_axis_name="s")


def _make_deg_kernel(KA, KB, SLOW):
    DW = 16
    K = KA + KB

    @functools.partial(
        pl.kernel,
        out_type=jax.ShapeDtypeStruct((NC, NPAD, DW), jnp.float32),
        mesh=_mesh(),
        compiler_params=pltpu.CompilerParams(use_tc_tiling_on_sc=False),
        scratch_types=[
            pltpu.VMEM((K, C), jnp.int32),
            pltpu.VMEM((C, DW), jnp.float32),
            pltpu.VMEM_SHARED((NPAD, DW), jnp.float32),
        ],
    )
    def deg_kernel(ea_hbm, eb_hbm, ones_hbm, zeros_hbm, out_hbm,
                   dst_v, ones_v, acc_sh):
        c = lax.axis_index("c")
        s = lax.axis_index("s")
        kmax = K if SLOW is None else jnp.where(c == SLOW, KA, K)
        pltpu.sync_copy(zeros_hbm.at[pl.ds(s * ZR, ZR)],
                        acc_sh.at[pl.ds(s * ZR, ZR)])
        pltpu.sync_copy(ea_hbm.at[1, :, c, s], dst_v.at[pl.ds(0, KA)])
        pltpu.sync_copy(eb_hbm.at[1, :, c, s], dst_v.at[pl.ds(KA, KB)])
        pltpu.sync_copy(ones_hbm, ones_v)
        plsc.subcore_barrier()

        def body(j, carry):
            @pl.when(j < kmax)
            def _():
                pltpu.sync_copy(ones_v, acc_sh.at[dst_v.at[j]], add=True)
            return carry

        lax.fori_loop(0, K, body, 0)
        plsc.subcore_barrier()
        pltpu.sync_copy(acc_sh.at[pl.ds(s * ZR, ZR)],
                        out_hbm.at[c].at[pl.ds(s * ZR, ZR)])

    return deg_kernel


def _make_prop_kernel(KA, KB, SLOW, W):
    K = KA + KB

    @functools.partial(
        pl.kernel,
        out_type=jax.ShapeDtypeStruct((NC, NPAD, W), jnp.float32),
        mesh=_mesh(),
        compiler_params=pltpu.CompilerParams(use_tc_tiling_on_sc=False),
        scratch_types=[
            pltpu.VMEM((K, C), jnp.int32),
            pltpu.VMEM((K, C), jnp.int32),
            pltpu.VMEM((C, W), jnp.float32),
            pltpu.VMEM((C, W), jnp.float32),
            pltpu.VMEM_SHARED((NPAD, W), jnp.float32),
            pltpu.SemaphoreType.DMA,
            pltpu.SemaphoreType.DMA,
        ],
    )
    def prop_kernel(hs_hbm, ea_hbm, eb_hbm, zeros_hbm, out_hbm,
                    src_v, dst_v, rows_a, rows_b, acc_sh, sem_a, sem_b):
        c = lax.axis_index("c")
        s = lax.axis_index("s")
        kmax = K if SLOW is None else jnp.where(c == SLOW, KA, K)
        pltpu.sync_copy(zeros_hbm.at[pl.ds(s * ZR, ZR)],
                        acc_sh.at[pl.ds(s * ZR, ZR)])
        pltpu.sync_copy(ea_hbm.at[0, :, c, s], src_v.at[pl.ds(0, KA)])
        pltpu.sync_copy(eb_hbm.at[0, :, c, s], src_v.at[pl.ds(KA, KB)])
        pltpu.sync_copy(ea_hbm.at[1, :, c, s], dst_v.at[pl.ds(0, KA)])
        pltpu.sync_copy(eb_hbm.at[1, :, c, s], dst_v.at[pl.ds(KA, KB)])
        plsc.subcore_barrier()

        pltpu.async_copy(hs_hbm.at[src_v.at[0]], rows_a, sem_a)

        def body2(i, carry):
            j = 2 * i

            @pl.when(j + 1 < kmax)
            def _():
                pltpu.async_copy(hs_hbm.at[src_v.at[j + 1]], rows_b, sem_b)

            @pl.when(j < kmax)
            def _():
                pltpu.make_async_copy(
                    hs_hbm.at[src_v.at[j]], rows_a, sem_a).wait()
                pltpu.sync_copy(rows_a, acc_sh.at[dst_v.at[j]], add=True)

            @pl.when(j + 2 < kmax)
            def _():
                pltpu.async_copy(hs_hbm.at[src_v.at[j + 2]], rows_a, sem_a)

            @pl.when(j + 1 < kmax)
            def _():
                pltpu.make_async_copy(
                    hs_hbm.at[src_v.at[j + 1]], rows_b, sem_b).wait()
                pltpu.sync_copy(rows_b, acc_sh.at[dst_v.at[j + 1]], add=True)

            return carry

        lax.fori_loop(0, (K + 1) // 2, body2, 0)
        plsc.subcore_barrier()
        pltpu.sync_copy(acc_sh.at[pl.ds(s * ZR, ZR)],
                        out_hbm.at[c].at[pl.ds(s * ZR, ZR)])

    return prop_kernel


def _row_mask():
    rows = lax.broadcasted_iota(jnp.int32, (NPAD, 1), 0)
    return rows < N


def _stage_a0(x_ref, w_ref, u_ref):
    u_ref[...] = jnp.dot(x_ref[...], w_ref[...],
                         preferred_element_type=jnp.float32)


def _stage_a1(u_ref, degp_ref, hs_ref, dis_ref):
    deg = degp_ref[0][:, 0:1] + degp_ref[1][:, 0:1]
    valid = _row_mask() & (deg > 0.0)
    dis = jnp.where(valid, lax.rsqrt(jnp.maximum(deg, 1e-20)), 0.0)
    dis_ref[...] = dis
    hs_ref[...] = u_ref[...] * dis


def _stage_b(narrow_out):
    def body(accq_ref, dis2_ref, b_ref, g_ref, be_ref, wblk_ref, hs_ref):
        dis2 = dis2_ref[...]
        mf = (dis2 > 0.0).astype(jnp.float32)
        h = ((accq_ref[0] + accq_ref[1]) * dis2 + b_ref[...]) * mf
        s = jnp.sum(h, axis=0, keepdims=True)
        mu = jnp.concatenate([(s[:, 0:H] + s[:, H:2 * H]) * (1.0 / N)] * 2,
                             axis=1)
        d = (h - mu) * mf
        v = jnp.sum(d * d, axis=0, keepdims=True)
        var = jnp.concatenate([(v[:, 0:H] + v[:, H:2 * H]) * (1.0 / N)] * 2,
                              axis=1)
        y = (h - mu) * lax.rsqrt(var + EPS) * g_ref[...] + be_ref[...]
        y = jnp.maximum(y, 0.0) * mf
        out = jnp.dot(y, wblk_ref[...], preferred_element_type=jnp.float32)
        if narrow_out:
            disn = jnp.concatenate([dis2[:, 0:H2], dis2[:, H:H + H2]], axis=1)
            hs_ref[...] = out * disn
        else:
            hs_ref[...] = out * dis2

    return body


def _c4(v, op):
    return op(op(v[:, 0:H2], v[:, H2:2 * H2]),
              op(v[:, 2 * H2:3 * H2], v[:, 3 * H2:4 * H2]))


def _stage_c(accq_ref, disq_ref, b4_ref, g4_ref, be4_ref, bid128_ref,
             bid_ref, wc1a_ref, wc1b_ref, bc1_ref, wc2_ref, bc2_ref,
             out_ref, sums_ref, mx_ref):
    b128 = bid128_ref[...]
    mf = (b128 < G).astype(jnp.float32)
    h = ((accq_ref[0] + accq_ref[1]) * disq_ref[...] + b4_ref[...]) * mf
    s = jnp.sum(h, axis=0, keepdims=True)
    mu = jnp.concatenate([_c4(s, jnp.add) * (1.0 / N)] * 4, axis=1)
    d = (h - mu) * mf
    v = jnp.sum(d * d, axis=0, keepdims=True)
    var = jnp.concatenate([_c4(v, jnp.add) * (1.0 / N)] * 4, axis=1)
    y = (h - mu) * lax.rsqrt(var + EPS) * g4_ref[...] + be4_ref[...]
    y = jnp.maximum(y, 0.0) * mf

    bid = bid_ref[...]
    oh = (bid == lax.broadcasted_iota(jnp.int32, (1, G), 1)).astype(jnp.float32)
    mcol = (bid < G).astype(jnp.float32)
    counts = lax.dot_general(oh, mcol, (((0,), (0,)), ((), ())),
                             preferred_element_type=jnp.float32)

    def gbody(g, carry):
        ym = jnp.where(b128 == g, y, 0.0)
        sg = jnp.sum(ym, axis=0, keepdims=True)
        mg = jnp.max(ym, axis=0, keepdims=True)
        sums_ref[pl.ds(g, 1), :] = _c4(sg, jnp.add)
        mx_ref[pl.ds(g, 1), :] = _c4(mg, jnp.maximum)
        return carry

    lax.fori_loop(0, G, gbody, 0)
    mean = sums_ref[...] / jnp.maximum(counts, 1.0)
    mx = mx_ref[...]

    zz = (lax.dot_general(mean, wc1a_ref[...], (((1,), (0,)), ((), ())),
                          preferred_element_type=jnp.float32)
          + lax.dot_general(mx, wc1b_ref[...], (((1,), (0,)), ((), ())),
                            preferred_element_type=jnp.float32)
          + bc1_ref[...])
    zz = jnp.maximum(zz, 0.0)
    out_ref[...] = lax.dot_general(zz, wc2_ref[...], (((1,), (0,)), ((), ())),
                                   preferred_element_type=jnp.float32) + bc2_ref[...]


def kernel(x, edge_index, batch, W1, b1, g1, be1, W2, b2, g2, be2,
           W3, b3, g3, be3, Wc1, bc1, Wc2, bc2):
    E = edge_index.shape[1]
    ET = E + N

    KA, KB, SLOW = 66, 15, None
    NEA = NC * NS * KA * C
    NEB = NC * NS * KB * C
    PADLEN = NEA + NEB - ET

    ei = edge_index.astype(jnp.int32)
    sl = jnp.arange(N, dtype=jnp.int32)
    loops = jnp.stack([sl, sl])
    padblk = jnp.stack([jnp.zeros((PADLEN,), jnp.int32),
                        N + (jnp.arange(PADLEN, dtype=jnp.int32) %
                             (NPAD - N))])
    alle = jnp.concatenate([ei, loops, padblk], axis=1)
    eA = alle[:, :NEA].reshape(2, KA, NC, NS, C)
    eB = alle[:, NEA:].reshape(2, KB, NC, NS, C)

    xp = jnp.pad(x, ((0, NPAD - N), (0, 0)))
    bidp = jnp.pad(batch.astype(jnp.int32), (0, NPAD - N),
                   constant_values=G).reshape(NPAD, 1)

    ones16 = jnp.ones((C, 16), jnp.float32)
    zeros16 = jnp.zeros((NPAD, 16), jnp.float32)
    zeros64 = jnp.zeros((NPAD, H), jnp.float32)
    zeros32 = jnp.zeros((NPAD, H2), jnp.float32)

    degp = _make_deg_kernel(KA, KB, SLOW)(eA, eB, ones16, zeros16)
    u1 = pl.pallas_call(
        _stage_a0,
        out_shape=jax.ShapeDtypeStruct((NPAD, H), jnp.float32),
    )(xp, W1)

    hs1, dis = pl.pallas_call(
        _stage_a1,
        out_shape=(jax.ShapeDtypeStruct((NPAD, H), jnp.float32),
                   jax.ShapeDtypeStruct((NPAD, 1), jnp.float32)),
    )(u1, degp)

    prop64 = _make_prop_kernel(KA, KB, SLOW, H)
    prop32 = _make_prop_kernel(KA, KB, SLOW, H2)

    NP2 = NPAD // 2
    dis2 = jnp.broadcast_to(dis, (NPAD, H)).reshape(NP2, 2 * H)
    zb = jnp.zeros((H, H), jnp.float32)
    w2blk = jnp.concatenate(
        [jnp.concatenate([W2, zb], 1), jnp.concatenate([zb, W2], 1)], 0)
    zb2 = jnp.zeros((H, H2), jnp.float32)
    w3blk = jnp.concatenate(
        [jnp.concatenate([W3, zb2], 1), jnp.concatenate([zb2, W3], 1)], 0)

    def b_call(accp, b, g, be, wblk, wout, narrow):
        return pl.pallas_call(
            _stage_b(narrow),
            out_shape=jax.ShapeDtypeStruct((NP2, wout), jnp.float32),
        )(accp.reshape(NC, NP2, 2 * H), dis2,
          jnp.tile(b.reshape(1, H), (1, 2)), jnp.tile(g.reshape(1, H), (1, 2)),
          jnp.tile(be.reshape(1, H), (1, 2)), wblk)

    accp1 = prop64(hs1, eA, eB, zeros64)
    hs2 = b_call(accp1, b1, g1, be1, w2blk, 2 * H, False).reshape(NPAD, H)

    accp2 = prop64(hs2, eA, eB, zeros64)
    hs3 = b_call(accp2, b2, g2, be2, w3blk, 2 * H2, True).reshape(NPAD, H2)

    accp3 = prop32(hs3, eA, eB, zeros32)

    NQ = NPAD // 4
    accq3 = accp3.reshape(NC, NQ, 4 * H2)
    disq = jnp.broadcast_to(dis, (NPAD, H2)).reshape(NQ, 4 * H2)
    bid128 = jnp.broadcast_to(bidp, (NPAD, H2)).reshape(NQ, 4 * H2)
    b4 = jnp.tile(b3.reshape(1, H2), (1, 4))
    g4 = jnp.tile(g3.reshape(1, H2), (1, 4))
    be4 = jnp.tile(be3.reshape(1, H2), (1, 4))

    out = pl.pallas_call(
        _stage_c,
        out_shape=jax.ShapeDtypeStruct((G, OUT), jnp.float32),
        scratch_shapes=[pltpu.VMEM((G, H2), jnp.float32),
                        pltpu.VMEM((G, H2), jnp.float32)],
    )(accq3, disq, b4, g4, be4, bid128, bidp,
      Wc1[:H2], Wc1[H2:], bc1.reshape(1, H2), Wc2, bc2.reshape(1, OUT))

    return out


# speedup vs baseline: 1.0894x; 1.0280x over previous
import functools

import jax
import jax.numpy as jnp
from jax import lax
from jax.experimental import pallas as pl
from jax.experimental.pallas import tpu as pltpu
from jax.experimental.pallas import tpu_sc as plsc

N = 10000
G = 64
DIN = 128
H = 64
H2 = 32
OUT = 2

NPAD = 10112
ZR = NPAD // 16
C = 128
NC, NS = 2, 16
EPS = 1e-5


def _mesh():
    return plsc.VectorSubcoreMesh(core_axis_name="c", subcore_axis_name="s")


def _make_deg_kernel(K):
    DW = 16

    @functools.partial(
        pl.kernel,
        out_type=jax.ShapeDtypeStruct((NC, NPAD, DW), jnp.float32),
        mesh=_mesh(),
        compiler_params=pltpu.CompilerParams(use_tc_tiling_on_sc=False),
        scratch_types=[
            pltpu.VMEM((K, C), jnp.int32),
            pltpu.VMEM((C, DW), jnp.float32),
            pltpu.VMEM_SHARED((NPAD, DW), jnp.float32),
        ],
    )
    def deg_kernel(edge_hbm, ones_hbm, zeros_hbm, out_hbm,
                   dst_v, ones_v, acc_sh):
        c = lax.axis_index("c")
        s = lax.axis_index("s")
        pltpu.sync_copy(zeros_hbm.at[pl.ds(s * ZR, ZR)],
                        acc_sh.at[pl.ds(s * ZR, ZR)])
        pltpu.sync_copy(edge_hbm.at[1, :, c, s], dst_v)
        pltpu.sync_copy(ones_hbm, ones_v)
        plsc.subcore_barrier()

        def body(j, carry):
            pltpu.sync_copy(ones_v, acc_sh.at[dst_v.at[j]], add=True)
            return carry

        lax.fori_loop(0, K, body, 0)
        plsc.subcore_barrier()
        pltpu.sync_copy(acc_sh.at[pl.ds(s * ZR, ZR)],
                        out_hbm.at[c].at[pl.ds(s * ZR, ZR)])

    return deg_kernel


def _make_prop_kernel(K, W):
    @functools.partial(
        pl.kernel,
        out_type=jax.ShapeDtypeStruct((NC, NPAD, W), jnp.float32),
        mesh=_mesh(),
        compiler_params=pltpu.CompilerParams(use_tc_tiling_on_sc=False),
        scratch_types=[
            pltpu.VMEM((K, C), jnp.int32),
            pltpu.VMEM((K, C), jnp.int32),
            pltpu.VMEM((C, W), jnp.float32),
            pltpu.VMEM((C, W), jnp.float32),
            pltpu.VMEM_SHARED((NPAD, W), jnp.float32),
            pltpu.SemaphoreType.DMA,
            pltpu.SemaphoreType.DMA,
        ],
    )
    def prop_kernel(hs_hbm, edge_hbm, zeros_hbm, out_hbm,
                    src_v, dst_v, rows_a, rows_b, acc_sh, sem_a, sem_b):
        c = lax.axis_index("c")
        s = lax.axis_index("s")
        pltpu.sync_copy(zeros_hbm.at[pl.ds(s * ZR, ZR)],
                        acc_sh.at[pl.ds(s * ZR, ZR)])
        pltpu.sync_copy(edge_hbm.at[0, :, c, s], src_v)
        pltpu.sync_copy(edge_hbm.at[1, :, c, s], dst_v)
        plsc.subcore_barrier()

        pltpu.async_copy(hs_hbm.at[src_v.at[0]], rows_a, sem_a)

        def body2(i, carry):
            j = 2 * i

            @pl.when(j + 1 < K)
            def _():
                pltpu.async_copy(hs_hbm.at[src_v.at[j + 1]], rows_b, sem_b)

            pltpu.make_async_copy(hs_hbm.at[src_v.at[j]], rows_a, sem_a).wait()
            pltpu.sync_copy(rows_a, acc_sh.at[dst_v.at[j]], add=True)

            @pl.when(j + 2 < K)
            def _():
                pltpu.async_copy(hs_hbm.at[src_v.at[j + 2]], rows_a, sem_a)

            @pl.when(j + 1 < K)
            def _():
                pltpu.make_async_copy(
                    hs_hbm.at[src_v.at[j + 1]], rows_b, sem_b).wait()
                pltpu.sync_copy(rows_b, acc_sh.at[dst_v.at[j + 1]], add=True)

            return carry

        lax.fori_loop(0, (K + 1) // 2, body2, 0)
        plsc.subcore_barrier()
        pltpu.sync_copy(acc_sh.at[pl.ds(s * ZR, ZR)],
                        out_hbm.at[c].at[pl.ds(s * ZR, ZR)])

    return prop_kernel


def _row_mask():
    rows = lax.broadcasted_iota(jnp.int32, (NPAD, 1), 0)
    return rows < N


def _stage_a0(x_ref, w_ref, u_ref):
    u_ref[...] = jnp.dot(x_ref[...], w_ref[...],
                         preferred_element_type=jnp.float32)


def _stage_a1(u_ref, degp_ref, hs_ref, dis_ref):
    deg = degp_ref[0][:, 0:1] + degp_ref[1][:, 0:1]
    valid = _row_mask() & (deg > 0.0)
    dis = jnp.where(valid, lax.rsqrt(jnp.maximum(deg, 1e-20)), 0.0)
    dis_ref[...] = dis
    hs_ref[...] = u_ref[...] * dis


def _stage_b(narrow_out):
    def body(accq_ref, dis2_ref, b_ref, g_ref, be_ref, wblk_ref, hs_ref):
        dis2 = dis2_ref[...]
        mf = (dis2 > 0.0).astype(jnp.float32)
        h = ((accq_ref[0] + accq_ref[1]) * dis2 + b_ref[...]) * mf
        s = jnp.sum(h, axis=0, keepdims=True)
        mu = jnp.concatenate([(s[:, 0:H] + s[:, H:2 * H]) * (1.0 / N)] * 2,
                             axis=1)
        d = (h - mu) * mf
        v = jnp.sum(d * d, axis=0, keepdims=True)
        var = jnp.concatenate([(v[:, 0:H] + v[:, H:2 * H]) * (1.0 / N)] * 2,
                              axis=1)
        y = (h - mu) * lax.rsqrt(var + EPS) * g_ref[...] + be_ref[...]
        y = jnp.maximum(y, 0.0) * mf
        out = jnp.dot(y, wblk_ref[...], preferred_element_type=jnp.float32)
        if narrow_out:
            disn = jnp.concatenate([dis2[:, 0:H2], dis2[:, H:H + H2]], axis=1)
            hs_ref[...] = out * disn
        else:
            hs_ref[...] = out * dis2

    return body


def _c4(v, op):
    return op(op(v[:, 0:H2], v[:, H2:2 * H2]),
              op(v[:, 2 * H2:3 * H2], v[:, 3 * H2:4 * H2]))


def _stage_c(accq_ref, disq_ref, b4_ref, g4_ref, be4_ref, bid128_ref,
             bid_ref, wc1a_ref, wc1b_ref, bc1_ref, wc2_ref, bc2_ref,
             out_ref, sums_ref, mx_ref):
    b128 = bid128_ref[...]
    mf = (b128 < G).astype(jnp.float32)
    h = ((accq_ref[0] + accq_ref[1]) * disq_ref[...] + b4_ref[...]) * mf
    s = jnp.sum(h, axis=0, keepdims=True)
    mu = jnp.concatenate([_c4(s, jnp.add) * (1.0 / N)] * 4, axis=1)
    d = (h - mu) * mf
    v = jnp.sum(d * d, axis=0, keepdims=True)
    var = jnp.concatenate([_c4(v, jnp.add) * (1.0 / N)] * 4, axis=1)
    y = (h - mu) * lax.rsqrt(var + EPS) * g4_ref[...] + be4_ref[...]
    y = jnp.maximum(y, 0.0) * mf

    bid = bid_ref[...]
    oh = (bid == lax.broadcasted_iota(jnp.int32, (1, G), 1)).astype(jnp.float32)
    mcol = (bid < G).astype(jnp.float32)
    counts = lax.dot_general(oh, mcol, (((0,), (0,)), ((), ())),
                             preferred_element_type=jnp.float32)

    def gbody(g, carry):
        ym = jnp.where(b128 == g, y, 0.0)
        sg = jnp.sum(ym, axis=0, keepdims=True)
        mg = jnp.max(ym, axis=0, keepdims=True)
        sums_ref[pl.ds(g, 1), :] = _c4(sg, jnp.add)
        mx_ref[pl.ds(g, 1), :] = _c4(mg, jnp.maximum)
        return carry

    lax.fori_loop(0, G, gbody, 0)
    mean = sums_ref[...] / jnp.maximum(counts, 1.0)
    mx = mx_ref[...]

    zz = (lax.dot_general(mean, wc1a_ref[...], (((1,), (0,)), ((), ())),
                          preferred_element_type=jnp.float32)
          + lax.dot_general(mx, wc1b_ref[...], (((1,), (0,)), ((), ())),
                            preferred_element_type=jnp.float32)
          + bc1_ref[...])
    zz = jnp.maximum(zz, 0.0)
    out_ref[...] = lax.dot_general(zz, wc2_ref[...], (((1,), (0,)), ((), ())),
                                   preferred_element_type=jnp.float32) + bc2_ref[...]


def kernel(x, edge_index, batch, W1, b1, g1, be1, W2, b2, g2, be2,
           W3, b3, g3, be3, Wc1, bc1, Wc2, bc2):
    E = edge_index.shape[1]
    ET = E + N

    K = -(-ET // (NC * NS * C))
    PADLEN = NC * NS * K * C - ET

    ei = edge_index.astype(jnp.int32)
    sl = jnp.arange(N, dtype=jnp.int32)
    loops = jnp.stack([sl, sl])
    padblk = jnp.stack([jnp.zeros((PADLEN,), jnp.int32),
                        N + (jnp.arange(PADLEN, dtype=jnp.int32) %
                             (NPAD - N))])
    e4 = jnp.concatenate([ei, loops, padblk], axis=1).reshape(2, K, NC, NS, C)

    xp = jnp.pad(x, ((0, NPAD - N), (0, 0)))
    bidp = jnp.pad(batch.astype(jnp.int32), (0, NPAD - N),
                   constant_values=G).reshape(NPAD, 1)

    ones16 = jnp.ones((C, 16), jnp.float32)
    zeros16 = jnp.zeros((NPAD, 16), jnp.float32)
    zeros64 = jnp.zeros((NPAD, H), jnp.float32)
    zeros32 = jnp.zeros((NPAD, H2), jnp.float32)

    degp = _make_deg_kernel(K)(e4, ones16, zeros16)
    u1 = pl.pallas_call(
        _stage_a0,
        out_shape=jax.ShapeDtypeStruct((NPAD, H), jnp.float32),
    )(xp, W1)

    hs1, dis = pl.pallas_call(
        _stage_a1,
        out_shape=(jax.ShapeDtypeStruct((NPAD, H), jnp.float32),
                   jax.ShapeDtypeStruct((NPAD, 1), jnp.float32)),
    )(u1, degp)

    prop64 = _make_prop_kernel(K, H)
    prop32 = _make_prop_kernel(K, H2)

    NP2 = NPAD // 2
    dis2 = jnp.broadcast_to(dis, (NPAD, H)).reshape(NP2, 2 * H)
    zb = jnp.zeros((H, H), jnp.float32)
    w2blk = jnp.concatenate(
        [jnp.concatenate([W2, zb], 1), jnp.concatenate([zb, W2], 1)], 0)
    zb2 = jnp.zeros((H, H2), jnp.float32)
    w3blk = jnp.concatenate(
        [jnp.concatenate([W3, zb2], 1), jnp.concatenate([zb2, W3], 1)], 0)

    def b_call(accp, b, g, be, wblk, wout, narrow):
        return pl.pallas_call(
            _stage_b(narrow),
            out_shape=jax.ShapeDtypeStruct((NP2, wout), jnp.float32),
        )(accp.reshape(NC, NP2, 2 * H), dis2,
          jnp.tile(b.reshape(1, H), (1, 2)), jnp.tile(g.reshape(1, H), (1, 2)),
          jnp.tile(be.reshape(1, H), (1, 2)), wblk)

    accp1 = prop64(hs1, e4, zeros64)
    hs2 = b_call(accp1, b1, g1, be1, w2blk, 2 * H, False).reshape(NPAD, H)

    accp2 = prop64(hs2, e4, zeros64)
    hs3 = b_call(accp2, b2, g2, be2, w3blk, 2 * H2, True).reshape(NPAD, H2)

    accp3 = prop32(hs3, e4, zeros32)

    NQ = NPAD // 4
    accq3 = accp3.reshape(NC, NQ, 4 * H2)
    disq = jnp.broadcast_to(dis, (NPAD, H2)).reshape(NQ, 4 * H2)
    bid128 = jnp.broadcast_to(bidp, (NPAD, H2)).reshape(NQ, 4 * H2)
    b4 = jnp.tile(b3.reshape(1, H2), (1, 4))
    g4 = jnp.tile(g3.reshape(1, H2), (1, 4))
    be4 = jnp.tile(be3.reshape(1, H2), (1, 4))

    out = pl.pallas_call(
        _stage_c,
        out_shape=jax.ShapeDtypeStruct((G, OUT), jnp.float32),
        scratch_shapes=[pltpu.VMEM((G, H2), jnp.float32),
                        pltpu.VMEM((G, H2), jnp.float32)],
    )(accq3, disq, b4, g4, be4, bid128, bidp,
      Wc1[:H2], Wc1[H2:], bc1.reshape(1, H2), Wc2, bc2.reshape(1, OUT))

    return out


# speedup vs baseline: 1.0967x; 1.0068x over previous
import functools

import jax
import jax.numpy as jnp
from jax import lax
from jax.experimental import pallas as pl
from jax.experimental.pallas import tpu as pltpu
from jax.experimental.pallas import tpu_sc as plsc

N = 10000
G = 64
DIN = 128
H = 64
H2 = 32
OUT = 2

NPAD = 10112
ZR = NPAD // 16
C = 128
NC, NS = 2, 16
EPS = 1e-5


def _mesh():
    return plsc.VectorSubcoreMesh(core_axis_name="c", subcore_axis_name="s")


def _make_deg_kernel(K):
    DW = 8

    @functools.partial(
        pl.kernel,
        out_type=jax.ShapeDtypeStruct((NC, NPAD, DW), jnp.float32),
        mesh=_mesh(),
        compiler_params=pltpu.CompilerParams(use_tc_tiling_on_sc=False),
        scratch_types=[
            pltpu.VMEM((K, C), jnp.int32),
            pltpu.VMEM((C, DW), jnp.float32),
            pltpu.VMEM_SHARED((NPAD, DW), jnp.float32),
        ],
    )
    def deg_kernel(edge_hbm, ones_hbm, zeros_hbm, out_hbm,
                   dst_v, ones_v, acc_sh):
        c = lax.axis_index("c")
        s = lax.axis_index("s")
        pltpu.sync_copy(zeros_hbm.at[pl.ds(s * ZR, ZR)],
                        acc_sh.at[pl.ds(s * ZR, ZR)])
        pltpu.sync_copy(edge_hbm.at[1, :, c, s], dst_v)
        pltpu.sync_copy(ones_hbm, ones_v)
        plsc.subcore_barrier()

        def body(j, carry):
            pltpu.sync_copy(ones_v, acc_sh.at[dst_v.at[j]], add=True)
            return carry

        lax.fori_loop(0, K, body, 0)
        plsc.subcore_barrier()
        pltpu.sync_copy(acc_sh.at[pl.ds(s * ZR, ZR)],
                        out_hbm.at[c].at[pl.ds(s * ZR, ZR)])

    return deg_kernel


def _make_prop_kernel(K, W):
    @functools.partial(
        pl.kernel,
        out_type=jax.ShapeDtypeStruct((NC, NPAD, W), jnp.float32),
        mesh=_mesh(),
        compiler_params=pltpu.CompilerParams(use_tc_tiling_on_sc=False),
        scratch_types=[
            pltpu.VMEM((K, C), jnp.int32),
            pltpu.VMEM((K, C), jnp.int32),
            pltpu.VMEM((C, W), jnp.float32),
            pltpu.VMEM((C, W), jnp.float32),
            pltpu.VMEM_SHARED((NPAD, W), jnp.float32),
            pltpu.SemaphoreType.DMA,
            pltpu.SemaphoreType.DMA,
        ],
    )
    def prop_kernel(hs_hbm, edge_hbm, zeros_hbm, out_hbm,
                    src_v, dst_v, rows_a, rows_b, acc_sh, sem_a, sem_b):
        c = lax.axis_index("c")
        s = lax.axis_index("s")
        pltpu.sync_copy(zeros_hbm.at[pl.ds(s * ZR, ZR)],
                        acc_sh.at[pl.ds(s * ZR, ZR)])
        pltpu.sync_copy(edge_hbm.at[0, :, c, s], src_v)
        pltpu.sync_copy(edge_hbm.at[1, :, c, s], dst_v)
        plsc.subcore_barrier()

        pltpu.async_copy(hs_hbm.at[src_v.at[0]], rows_a, sem_a)

        def body2(i, carry):
            j = 2 * i

            @pl.when(j + 1 < K)
            def _():
                pltpu.async_copy(hs_hbm.at[src_v.at[j + 1]], rows_b, sem_b)

            pltpu.make_async_copy(hs_hbm.at[src_v.at[j]], rows_a, sem_a).wait()
            pltpu.sync_copy(rows_a, acc_sh.at[dst_v.at[j]], add=True)

            @pl.when(j + 2 < K)
            def _():
                pltpu.async_copy(hs_hbm.at[src_v.at[j + 2]], rows_a, sem_a)

            @pl.when(j + 1 < K)
            def _():
                pltpu.make_async_copy(
                    hs_hbm.at[src_v.at[j + 1]], rows_b, sem_b).wait()
                pltpu.sync_copy(rows_b, acc_sh.at[dst_v.at[j + 1]], add=True)

            return carry

        lax.fori_loop(0, (K + 1) // 2, body2, 0)
        plsc.subcore_barrier()
        pltpu.sync_copy(acc_sh.at[pl.ds(s * ZR, ZR)],
                        out_hbm.at[c].at[pl.ds(s * ZR, ZR)])

    return prop_kernel


def _row_mask():
    rows = lax.broadcasted_iota(jnp.int32, (NPAD, 1), 0)
    return rows < N


def _stage_a0(x_ref, w_ref, u_ref):
    u_ref[...] = jnp.dot(x_ref[...], w_ref[...],
                         preferred_element_type=jnp.float32)


def _stage_a1(u_ref, degp_ref, hs_ref, dis_ref):
    deg = degp_ref[0][:, 0:1] + degp_ref[1][:, 0:1]
    valid = _row_mask() & (deg > 0.0)
    dis = jnp.where(valid, lax.rsqrt(jnp.maximum(deg, 1e-20)), 0.0)
    dis_ref[...] = dis
    hs_ref[...] = u_ref[...] * dis


def _stage_b(narrow_out):
    def body(accq_ref, dis2_ref, b_ref, g_ref, be_ref, wblk_ref, hs_ref):
        dis2 = dis2_ref[...]
        mf = (dis2 > 0.0).astype(jnp.float32)
        h = ((accq_ref[0] + accq_ref[1]) * dis2 + b_ref[...]) * mf
        s = jnp.sum(h, axis=0, keepdims=True)
        mu = jnp.concatenate([(s[:, 0:H] + s[:, H:2 * H]) * (1.0 / N)] * 2,
                             axis=1)
        d = (h - mu) * mf
        v = jnp.sum(d * d, axis=0, keepdims=True)
        var = jnp.concatenate([(v[:, 0:H] + v[:, H:2 * H]) * (1.0 / N)] * 2,
                              axis=1)
        y = (h - mu) * lax.rsqrt(var + EPS) * g_ref[...] + be_ref[...]
        y = jnp.maximum(y, 0.0) * mf
        out = jnp.dot(y, wblk_ref[...], preferred_element_type=jnp.float32)
        if narrow_out:
            disn = jnp.concatenate([dis2[:, 0:H2], dis2[:, H:H + H2]], axis=1)
            hs_ref[...] = out * disn
        else:
            hs_ref[...] = out * dis2

    return body


def _c4(v, op):
    return op(op(v[:, 0:H2], v[:, H2:2 * H2]),
              op(v[:, 2 * H2:3 * H2], v[:, 3 * H2:4 * H2]))


def _stage_c(accq_ref, disq_ref, b4_ref, g4_ref, be4_ref, bid128_ref,
             bid_ref, wc1a_ref, wc1b_ref, bc1_ref, wc2_ref, bc2_ref,
             out_ref, sums_ref, mx_ref):
    b128 = bid128_ref[...]
    mf = (b128 < G).astype(jnp.float32)
    h = ((accq_ref[0] + accq_ref[1]) * disq_ref[...] + b4_ref[...]) * mf
    s = jnp.sum(h, axis=0, keepdims=True)
    mu = jnp.concatenate([_c4(s, jnp.add) * (1.0 / N)] * 4, axis=1)
    d = (h - mu) * mf
    v = jnp.sum(d * d, axis=0, keepdims=True)
    var = jnp.concatenate([_c4(v, jnp.add) * (1.0 / N)] * 4, axis=1)
    y = (h - mu) * lax.rsqrt(var + EPS) * g4_ref[...] + be4_ref[...]
    y = jnp.maximum(y, 0.0) * mf

    bid = bid_ref[...]
    oh = (bid == lax.broadcasted_iota(jnp.int32, (1, G), 1)).astype(jnp.float32)
    mcol = (bid < G).astype(jnp.float32)
    counts = lax.dot_general(oh, mcol, (((0,), (0,)), ((), ())),
                             preferred_element_type=jnp.float32)

    def gbody(g, carry):
        ym = jnp.where(b128 == g, y, 0.0)
        sg = jnp.sum(ym, axis=0, keepdims=True)
        mg = jnp.max(ym, axis=0, keepdims=True)
        sums_ref[pl.ds(g, 1), :] = _c4(sg, jnp.add)
        mx_ref[pl.ds(g, 1), :] = _c4(mg, jnp.maximum)
        return carry

    lax.fori_loop(0, G, gbody, 0)
    mean = sums_ref[...] / jnp.maximum(counts, 1.0)
    mx = mx_ref[...]

    zz = (lax.dot_general(mean, wc1a_ref[...], (((1,), (0,)), ((), ())),
                          preferred_element_type=jnp.float32)
          + lax.dot_general(mx, wc1b_ref[...], (((1,), (0,)), ((), ())),
                            preferred_element_type=jnp.float32)
          + bc1_ref[...])
    zz = jnp.maximum(zz, 0.0)
    out_ref[...] = lax.dot_general(zz, wc2_ref[...], (((1,), (0,)), ((), ())),
                                   preferred_element_type=jnp.float32) + bc2_ref[...]


def kernel(x, edge_index, batch, W1, b1, g1, be1, W2, b2, g2, be2,
           W3, b3, g3, be3, Wc1, bc1, Wc2, bc2):
    E = edge_index.shape[1]
    ET = E + N

    K = -(-ET // (NC * NS * C))
    PADLEN = NC * NS * K * C - ET

    ei = edge_index.astype(jnp.int32)
    sl = jnp.arange(N, dtype=jnp.int32)
    loops = jnp.stack([sl, sl])
    padblk = jnp.stack([jnp.zeros((PADLEN,), jnp.int32),
                        N + (jnp.arange(PADLEN, dtype=jnp.int32) %
                             (NPAD - N))])
    e4 = jnp.concatenate([ei, loops, padblk], axis=1).reshape(2, K, NC, NS, C)

    xp = jnp.pad(x, ((0, NPAD - N), (0, 0)))
    bidp = jnp.pad(batch.astype(jnp.int32), (0, NPAD - N),
                   constant_values=G).reshape(NPAD, 1)

    ones16 = jnp.ones((C, 8), jnp.float32)
    zeros16 = jnp.zeros((NPAD, 8), jnp.float32)
    zeros64 = jnp.zeros((NPAD, H), jnp.float32)
    zeros32 = jnp.zeros((NPAD, H2), jnp.float32)

    degp = _make_deg_kernel(K)(e4, ones16, zeros16)
    u1 = pl.pallas_call(
        _stage_a0,
        out_shape=jax.ShapeDtypeStruct((NPAD, H), jnp.float32),
    )(xp, W1)

    hs1, dis = pl.pallas_call(
        _stage_a1,
        out_shape=(jax.ShapeDtypeStruct((NPAD, H), jnp.float32),
                   jax.ShapeDtypeStruct((NPAD, 1), jnp.float32)),
    )(u1, degp)

    prop64 = _make_prop_kernel(K, H)
    prop32 = _make_prop_kernel(K, H2)

    NP2 = NPAD // 2
    dis2 = jnp.broadcast_to(dis, (NPAD, H)).reshape(NP2, 2 * H)
    zb = jnp.zeros((H, H), jnp.float32)
    w2blk = jnp.concatenate(
        [jnp.concatenate([W2, zb], 1), jnp.concatenate([zb, W2], 1)], 0)
    zb2 = jnp.zeros((H, H2), jnp.float32)
    w3blk = jnp.concatenate(
        [jnp.concatenate([W3, zb2], 1), jnp.concatenate([zb2, W3], 1)], 0)

    def b_call(accp, b, g, be, wblk, wout, narrow):
        return pl.pallas_call(
            _stage_b(narrow),
            out_shape=jax.ShapeDtypeStruct((NP2, wout), jnp.float32),
        )(accp.reshape(NC, NP2, 2 * H), dis2,
          jnp.tile(b.reshape(1, H), (1, 2)), jnp.tile(g.reshape(1, H), (1, 2)),
          jnp.tile(be.reshape(1, H), (1, 2)), wblk)

    accp1 = prop64(hs1, e4, zeros64)
    hs2 = b_call(accp1, b1, g1, be1, w2blk, 2 * H, False).reshape(NPAD, H)

    accp2 = prop64(hs2, e4, zeros64)
    hs3 = b_call(accp2, b2, g2, be2, w3blk, 2 * H2, True).reshape(NPAD, H2)

    accp3 = prop32(hs3, e4, zeros32)

    NQ = NPAD // 4
    accq3 = accp3.reshape(NC, NQ, 4 * H2)
    disq = jnp.broadcast_to(dis, (NPAD, H2)).reshape(NQ, 4 * H2)
    bid128 = jnp.broadcast_to(bidp, (NPAD, H2)).reshape(NQ, 4 * H2)
    b4 = jnp.tile(b3.reshape(1, H2), (1, 4))
    g4 = jnp.tile(g3.reshape(1, H2), (1, 4))
    be4 = jnp.tile(be3.reshape(1, H2), (1, 4))

    out = pl.pallas_call(
        _stage_c,
        out_shape=jax.ShapeDtypeStruct((G, OUT), jnp.float32),
        scratch_shapes=[pltpu.VMEM((G, H2), jnp.float32),
                        pltpu.VMEM((G, H2), jnp.float32)],
    )(accq3, disq, b4, g4, be4, bid128, bidp,
      Wc1[:H2], Wc1[H2:], bc1.reshape(1, H2), Wc2, bc2.reshape(1, OUT))

    return out
